# baseline (speedup 1.0000x reference)
import functools

import jax
import jax.numpy as jnp
from jax import lax
from jax.experimental import pallas as pl
from jax.experimental.pallas import tpu as pltpu
from jax.experimental.pallas import tpu_sc as plsc

_EPS = 1e-6
_TB = 256
_TBS = 256
_FB = 512
_NW = 32


def _router_body(xt_ref, rms_ref, rw_ref, rb_ref, xn_ref, idx_ref, aff_ref):
    x = xt_ref[:]
    var = jnp.mean(x * x, axis=1, keepdims=True)
    xn = (x * lax.rsqrt(var + _EPS)) * rms_ref[:]
    xn_ref[:] = xn
    logits = jnp.dot(xn, rw_ref[:], preferred_element_type=jnp.float32)
    scores = jax.nn.sigmoid(logits)
    choice = scores + rb_ref[:]
    iota = lax.broadcasted_iota(jnp.int32, (_TB, 128), 1)
    m1 = jnp.max(choice, axis=1, keepdims=True)
    i1 = jnp.min(jnp.where(choice == m1, iota, 128), axis=1, keepdims=True)
    mask1 = iota == i1
    choice2 = jnp.where(mask1, -jnp.inf, choice)
    m2 = jnp.max(choice2, axis=1, keepdims=True)
    i2 = jnp.min(jnp.where(choice2 == m2, iota, 128), axis=1, keepdims=True)
    mask2 = iota == i2
    a1 = jnp.sum(jnp.where(mask1, scores, 0.0), axis=1, keepdims=True)
    a2 = jnp.sum(jnp.where(mask2, scores, 0.0), axis=1, keepdims=True)
    den = a1 + a2 + 1e-9
    idx_ref[:] = jnp.concatenate([i1, i2], axis=1)
    aff_ref[:] = jnp.concatenate([a1 / den, a2 / den], axis=1)


def _gemm_body(te_ref, act_ref, xs_ref, wc_ref, g_ref, u_ref, d_ref,
               eo_ref, gbf, ubf, dbf):
    t = pl.program_id(1)

    @pl.when(act_ref[t] == 1)
    def _compute():
        prev = te_ref[jnp.maximum(t - 1, 0)]

        @pl.when(jnp.logical_or(t == 0, prev != te_ref[t]))
        def _cast_weights():
            gbf[:] = g_ref[0].astype(jnp.bfloat16)
            ubf[:] = u_ref[0].astype(jnp.bfloat16)
            dbf[:] = d_ref[0].astype(jnp.bfloat16)

        xsb = xs_ref[:].astype(jnp.bfloat16)
        g = jnp.dot(xsb, gbf[:], preferred_element_type=jnp.float32)
        u = jnp.dot(xsb, ubf[:], preferred_element_type=jnp.float32)
        h = (g * jax.nn.sigmoid(g)) * u
        o = jnp.dot(h.astype(jnp.bfloat16), dbf[:],
                    preferred_element_type=jnp.float32)
        eo_ref[0] = o * wc_ref[:]


def _sc_gather(xn, row_token, p_rows, d):
    rows_w = p_rows // _NW
    chunk = 48
    nch = rows_w // chunk
    mesh = plsc.VectorSubcoreMesh(core_axis_name="c", subcore_axis_name="s")

    @functools.partial(
        pl.kernel, mesh=mesh,
        out_type=jax.ShapeDtypeStruct((p_rows, d), jnp.float32),
        scratch_types=[
            pltpu.VMEM((chunk,), jnp.int32),
            pltpu.VMEM((chunk, d), jnp.float32),
            pltpu.SemaphoreType.DMA,
        ],
    )
    def gk(xn_hbm, rt_hbm, xs_hbm, idx_v, rows_v, sem):
        wid = lax.axis_index("s") * 2 + lax.axis_index("c")
        for ch in range(nch):
            base = wid * rows_w + ch * chunk
            pltpu.sync_copy(rt_hbm.at[pl.ds(base, chunk)], idx_v)
            pltpu.async_copy(xn_hbm.at[idx_v], rows_v, sem).wait()
            pltpu.sync_copy(rows_v, xs_hbm.at[pl.ds(base, chunk)])

    return gk(xn, row_token)


def _sc_combine(xt, eof, idx4, t_tok, d):
    tok_w = t_tok // _NW
    ct = 16
    nch = tok_w // ct
    nj = idx4.shape[0]
    groups = ct * (d // 16)
    mesh = plsc.VectorSubcoreMesh(core_axis_name="c", subcore_axis_name="s")

    @functools.partial(
        pl.kernel, mesh=mesh,
        out_type=jax.ShapeDtypeStruct((t_tok, d), jnp.float32),
        scratch_types=[
            pltpu.VMEM((ct, d), jnp.float32),
            pltpu.VMEM((ct, d), jnp.float32),
            pltpu.VMEM((ct,), jnp.int32),
            pltpu.SemaphoreType.DMA,
        ],
    )
    def ck(xt_hbm, eof_hbm, idx_hbm, out_hbm, acc_v, buf_v, idx_v, sem):
        wid = lax.axis_index("s") * 2 + lax.axis_index("c")
        for ch in range(nch):
            base = wid * tok_w + ch * ct
            pltpu.sync_copy(xt_hbm.at[pl.ds(base, ct)], acc_v)
            for j in range(nj):
                pltpu.sync_copy(idx_hbm.at[j, pl.ds(base, ct)], idx_v)
                pltpu.async_copy(eof_hbm.at[idx_v], buf_v, sem).wait()

                def addbody(i, _):
                    r = i // (d // 16)
                    c = (i % (d // 16)) * 16
                    acc_v[r, pl.ds(c, 16)] = (acc_v[r, pl.ds(c, 16)]
                                              + buf_v[r, pl.ds(c, 16)])
                    return 0

                lax.fori_loop(0, groups, addbody, 0)
            pltpu.sync_copy(acc_v, out_hbm.at[pl.ds(base, ct)])

    return ck(xt, eof, idx4)


def kernel(hidden_states, rms_weight, router_weight, router_bias,
           gate_proj, up_proj, down_proj):
    b, s, d = hidden_states.shape
    e_num, _, f_dim = gate_proj.shape
    t_tok = b * s
    nt = t_tok // _TB
    nf = f_dim // _FB
    n_pairs = 2 * t_tok
    ntiles = (n_pairs + e_num * _TBS) // _TBS
    p_rows = ntiles * _TBS
    xt = hidden_states.reshape(t_tok, d)
    rw_pad = jnp.zeros((d, 128), jnp.float32).at[:, :e_num].set(router_weight.T)
    rb_pad = jnp.full((1, 128), -1e30, jnp.float32).at[0, :e_num].set(router_bias)
    rms2 = rms_weight.reshape(1, d)

    xn, idx, aff = pl.pallas_call(
        _router_body,
        grid=(nt,),
        in_specs=[
            pl.BlockSpec((_TB, d), lambda t: (t, 0)),
            pl.BlockSpec((1, d), lambda t: (0, 0)),
            pl.BlockSpec((d, 128), lambda t: (0, 0)),
            pl.BlockSpec((1, 128), lambda t: (0, 0)),
        ],
        out_specs=[
            pl.BlockSpec((_TB, d), lambda t: (t, 0)),
            pl.BlockSpec((_TB, 2), lambda t: (t, 0)),
            pl.BlockSpec((_TB, 2), lambda t: (t, 0)),
        ],
        out_shape=[
            jax.ShapeDtypeStruct((t_tok, d), jnp.float32),
            jax.ShapeDtypeStruct((t_tok, 2), jnp.int32),
            jax.ShapeDtypeStruct((t_tok, 2), jnp.float32),
        ],
    )(xt, rms2, rw_pad, rb_pad)

    flat_e = idx.reshape(n_pairs)
    flat_aff = aff.reshape(n_pairs)
    onehot = (flat_e[:, None] == jnp.arange(e_num)[None, :]).astype(jnp.int32)
    counts = jnp.sum(onehot, axis=0)
    rank = jnp.take_along_axis(jnp.cumsum(onehot, axis=0) - onehot,
                               flat_e[:, None], axis=1)[:, 0]
    p_pad = ((counts + _TBS - 1) // _TBS) * _TBS
    starts = jnp.concatenate([jnp.zeros(1, p_pad.dtype),
                              jnp.cumsum(p_pad)[:-1]])
    pos = (starts[flat_e] + rank).astype(jnp.int32)
    row_token = jnp.zeros(p_rows, jnp.int32).at[pos].set(
        jnp.arange(n_pairs, dtype=jnp.int32) // 2)
    w_sorted = jnp.zeros((p_rows, 1), jnp.float32).at[pos, 0].set(flat_aff)
    n_used = (jnp.sum(p_pad) // _TBS).astype(jnp.int32)
    tile_ids = jnp.arange(ntiles, dtype=jnp.int32)
    tile_e = jnp.clip(
        jnp.searchsorted(starts, tile_ids * _TBS, side='right') - 1,
        0, e_num - 1).astype(jnp.int32)
    tile_act = (tile_ids < n_used).astype(jnp.int32)
    te_last = tile_e[jnp.clip(n_used - 1, 0, ntiles - 1)]
    tile_e = jnp.where(tile_act == 1, tile_e, te_last)

    xs = _sc_gather(xn, row_token, p_rows, d)

    eo = pl.pallas_call(
        _gemm_body,
        grid_spec=pltpu.PrefetchScalarGridSpec(
            num_scalar_prefetch=2,
            grid=(nf, ntiles),
            in_specs=[
                pl.BlockSpec((_TBS, d), lambda f, t, te, act: (t, 0)),
                pl.BlockSpec((_TBS, 1), lambda f, t, te, act: (t, 0)),
                pl.BlockSpec((1, d, _FB), lambda f, t, te, act: (te[t], 0, f)),
                pl.BlockSpec((1, d, _FB), lambda f, t, te, act: (te[t], 0, f)),
                pl.BlockSpec((1, _FB, d), lambda f, t, te, act: (te[t], f, 0)),
            ],
            out_specs=pl.BlockSpec((1, _TBS, d),
                                   lambda f, t, te, act: (f, t, 0)),
            scratch_shapes=[
                pltpu.VMEM((d, _FB), jnp.bfloat16),
                pltpu.VMEM((d, _FB), jnp.bfloat16),
                pltpu.VMEM((_FB, d), jnp.bfloat16),
            ],
        ),
        out_shape=jax.ShapeDtypeStruct((nf, p_rows, d), jnp.float32),
        compiler_params=pltpu.CompilerParams(
            dimension_semantics=("arbitrary", "arbitrary")),
    )(tile_e, tile_act, xs, w_sorted, gate_proj, up_proj, down_proj)

    eof = eo.reshape(nf * p_rows, d)
    plane = jnp.arange(nf, dtype=jnp.int32)[:, None] * p_rows
    idx4 = (jnp.stack([pos[0::2], pos[1::2]])[None, :, :]
            + plane[:, :, None]).reshape(2 * nf, t_tok)

    out = _sc_combine(xt, eof, idx4, t_tok, d)
    return out.reshape(b, s, d)


# speedup vs baseline: 1.1669x; 1.1669x over previous
import functools

import jax
import jax.numpy as jnp
from jax import lax
from jax.experimental import pallas as pl
from jax.experimental.pallas import tpu as pltpu
from jax.experimental.pallas import tpu_sc as plsc

_EPS = 1e-6
_TB = 256
_TBS = 256
_FB = 512
_NW = 32


def _router_body(xt_ref, rms_ref, rw_ref, rb_ref, xn_ref, idx_ref, aff_ref):
    x = xt_ref[:]
    var = jnp.mean(x * x, axis=1, keepdims=True)
    xn = (x * lax.rsqrt(var + _EPS)) * rms_ref[:]
    xn_ref[:] = xn
    logits = jnp.dot(xn, rw_ref[:], preferred_element_type=jnp.float32)
    scores = jax.nn.sigmoid(logits)
    choice = scores + rb_ref[:]
    iota = lax.broadcasted_iota(jnp.int32, (_TB, 128), 1)
    m1 = jnp.max(choice, axis=1, keepdims=True)
    i1 = jnp.min(jnp.where(choice == m1, iota, 128), axis=1, keepdims=True)
    mask1 = iota == i1
    choice2 = jnp.where(mask1, -jnp.inf, choice)
    m2 = jnp.max(choice2, axis=1, keepdims=True)
    i2 = jnp.min(jnp.where(choice2 == m2, iota, 128), axis=1, keepdims=True)
    mask2 = iota == i2
    a1 = jnp.sum(jnp.where(mask1, scores, 0.0), axis=1, keepdims=True)
    a2 = jnp.sum(jnp.where(mask2, scores, 0.0), axis=1, keepdims=True)
    den = a1 + a2 + 1e-9
    idx_ref[:] = jnp.concatenate([i1, i2], axis=1)
    aff_ref[:] = jnp.concatenate([a1 / den, a2 / den], axis=1)


def _gemm_body(te_ref, act_ref, xs_ref, wc_ref, g_ref, u_ref, d_ref,
               eo_ref, gbf, ubf, dbf):
    t = pl.program_id(1)

    @pl.when(act_ref[t] == 1)
    def _compute():
        prev = te_ref[jnp.maximum(t - 1, 0)]

        @pl.when(jnp.logical_or(t == 0, prev != te_ref[t]))
        def _cast_weights():
            gbf[:] = g_ref[0].astype(jnp.bfloat16)
            ubf[:] = u_ref[0].astype(jnp.bfloat16)
            dbf[:] = d_ref[0].astype(jnp.bfloat16)

        xsb = xs_ref[:].astype(jnp.bfloat16)
        g = jnp.dot(xsb, gbf[:], preferred_element_type=jnp.float32)
        u = jnp.dot(xsb, ubf[:], preferred_element_type=jnp.float32)
        h = (g * jax.nn.sigmoid(g)) * u
        o = jnp.dot(h.astype(jnp.bfloat16), dbf[:],
                    preferred_element_type=jnp.float32)
        eo_ref[0] = o * wc_ref[:]


def _sc_gather(xn, row_token, p_rows, d):
    rows_w = p_rows // _NW
    chunk = 24
    nch = rows_w // chunk
    mesh = plsc.VectorSubcoreMesh(core_axis_name="c", subcore_axis_name="s")

    @functools.partial(
        pl.kernel, mesh=mesh,
        out_type=jax.ShapeDtypeStruct((p_rows, d), jnp.float32),
        scratch_types=[
            pltpu.VMEM((rows_w,), jnp.int32),
            pltpu.VMEM((chunk, d), jnp.float32),
            pltpu.VMEM((chunk, d), jnp.float32),
            pltpu.SemaphoreType.DMA,
            pltpu.SemaphoreType.DMA,
            pltpu.SemaphoreType.DMA,
        ],
    )
    def gk(xn_hbm, rt_hbm, xs_hbm, idx_v, rows_a, rows_b, gsem, ssem_a, ssem_b):
        wid = lax.axis_index("s") * 2 + lax.axis_index("c")
        base = wid * rows_w
        pltpu.sync_copy(rt_hbm.at[pl.ds(base, rows_w)], idx_v)
        bufs = (rows_a, rows_b)
        ssems = (ssem_a, ssem_b)
        g0 = pltpu.async_copy(xn_hbm.at[idx_v.at[pl.ds(0, chunk)]],
                              bufs[0], gsem)
        gathers = [g0]
        stores = [None, None]
        for ch in range(nch):
            nb = (ch + 1) % 2
            if ch + 1 < nch:
                if stores[nb] is not None:
                    stores[nb].wait()
                    stores[nb] = None
                gathers.append(pltpu.async_copy(
                    xn_hbm.at[idx_v.at[pl.ds((ch + 1) * chunk, chunk)]],
                    bufs[nb], gsem))
            gathers[ch].wait()
            stores[ch % 2] = pltpu.async_copy(
                bufs[ch % 2], xs_hbm.at[pl.ds(base + ch * chunk, chunk)],
                ssems[ch % 2])
        for st in stores:
            if st is not None:
                st.wait()

    return gk(xn, row_token)


def _sc_combine(xt, eof, idx4r, t_tok, d):
    tok_w = t_tok // _NW
    ct = 8
    nch = tok_w // ct
    nj = idx4r.shape[1]
    mesh = plsc.VectorSubcoreMesh(core_axis_name="c", subcore_axis_name="s")

    @functools.partial(
        pl.kernel, mesh=mesh,
        out_type=jax.ShapeDtypeStruct((t_tok, d), jnp.float32),
        scratch_types=[
            pltpu.VMEM((ct, d), jnp.float32),
            pltpu.VMEM((nj, ct, d), jnp.float32),
            pltpu.VMEM((nj, ct), jnp.int32),
            pltpu.SemaphoreType.DMA,
            pltpu.SemaphoreType.DMA,
        ],
    )
    def ck(xt_hbm, eof_hbm, idx_hbm, out_hbm, acc_v, bufs_v, idx_v, sem, osem):
        wid = lax.axis_index("s") * 2 + lax.axis_index("c")
        ost = None
        for ch in range(nch):
            base = wid * tok_w + ch * ct
            chid = wid * nch + ch
            pltpu.sync_copy(idx_hbm.at[chid], idx_v)
            cps = [pltpu.async_copy(
                eof_hbm.at[idx_v.at[j]], bufs_v.at[j], sem)
                for j in range(nj)]
            if ost is not None:
                ost.wait()
            cps.append(pltpu.async_copy(xt_hbm.at[pl.ds(base, ct)], acc_v, sem))
            for cp in cps:
                cp.wait()

            def addbody(i, _):
                r = i // (d // 64)
                c = (i % (d // 64)) * 64
                for q in range(4):
                    cq = c + q * 16
                    acc_v[r, pl.ds(cq, 16)] = (
                        ((acc_v[r, pl.ds(cq, 16)]
                          + bufs_v[0, r, pl.ds(cq, 16)])
                         + (bufs_v[1, r, pl.ds(cq, 16)]
                            + bufs_v[2, r, pl.ds(cq, 16)]))
                        + bufs_v[3, r, pl.ds(cq, 16)])
                return 0

            lax.fori_loop(0, ct * (d // 64), addbody, 0)
            ost = pltpu.async_copy(acc_v, out_hbm.at[pl.ds(base, ct)], osem)
        if ost is not None:
            ost.wait()

    return ck(xt, eof, idx4r)


def kernel(hidden_states, rms_weight, router_weight, router_bias,
           gate_proj, up_proj, down_proj):
    b, s, d = hidden_states.shape
    e_num, _, f_dim = gate_proj.shape
    t_tok = b * s
    nt = t_tok // _TB
    nf = f_dim // _FB
    n_pairs = 2 * t_tok
    ntiles = (n_pairs + e_num * _TBS) // _TBS
    p_rows = ntiles * _TBS
    xt = hidden_states.reshape(t_tok, d)
    rw_pad = jnp.zeros((d, 128), jnp.float32).at[:, :e_num].set(router_weight.T)
    rb_pad = jnp.full((1, 128), -1e30, jnp.float32).at[0, :e_num].set(router_bias)
    rms2 = rms_weight.reshape(1, d)

    xn, idx, aff = pl.pallas_call(
        _router_body,
        grid=(nt,),
        in_specs=[
            pl.BlockSpec((_TB, d), lambda t: (t, 0)),
            pl.BlockSpec((1, d), lambda t: (0, 0)),
            pl.BlockSpec((d, 128), lambda t: (0, 0)),
            pl.BlockSpec((1, 128), lambda t: (0, 0)),
        ],
        out_specs=[
            pl.BlockSpec((_TB, d), lambda t: (t, 0)),
            pl.BlockSpec((_TB, 2), lambda t: (t, 0)),
            pl.BlockSpec((_TB, 2), lambda t: (t, 0)),
        ],
        out_shape=[
            jax.ShapeDtypeStruct((t_tok, d), jnp.float32),
            jax.ShapeDtypeStruct((t_tok, 2), jnp.int32),
            jax.ShapeDtypeStruct((t_tok, 2), jnp.float32),
        ],
    )(xt, rms2, rw_pad, rb_pad)

    flat_e = idx.reshape(n_pairs)
    flat_aff = aff.reshape(n_pairs)
    onehot = (flat_e[:, None] == jnp.arange(e_num)[None, :]).astype(jnp.int32)
    counts = jnp.sum(onehot, axis=0)
    rank = jnp.take_along_axis(jnp.cumsum(onehot, axis=0) - onehot,
                               flat_e[:, None], axis=1)[:, 0]
    p_pad = ((counts + _TBS - 1) // _TBS) * _TBS
    starts = jnp.concatenate([jnp.zeros(1, p_pad.dtype),
                              jnp.cumsum(p_pad)[:-1]])
    pos = (starts[flat_e] + rank).astype(jnp.int32)
    row_token = jnp.zeros(p_rows, jnp.int32).at[pos].set(
        jnp.arange(n_pairs, dtype=jnp.int32) // 2)
    w_sorted = jnp.zeros((p_rows, 1), jnp.float32).at[pos, 0].set(flat_aff)
    n_used = (jnp.sum(p_pad) // _TBS).astype(jnp.int32)
    tile_ids = jnp.arange(ntiles, dtype=jnp.int32)
    tile_e = jnp.clip(
        jnp.searchsorted(starts, tile_ids * _TBS, side='right') - 1,
        0, e_num - 1).astype(jnp.int32)
    tile_act = (tile_ids < n_used).astype(jnp.int32)
    te_last = tile_e[jnp.clip(n_used - 1, 0, ntiles - 1)]
    tile_e = jnp.where(tile_act == 1, tile_e, te_last)

    xs = _sc_gather(xn, row_token, p_rows, d)

    eo = pl.pallas_call(
        _gemm_body,
        grid_spec=pltpu.PrefetchScalarGridSpec(
            num_scalar_prefetch=2,
            grid=(nf, ntiles),
            in_specs=[
                pl.BlockSpec((_TBS, d), lambda f, t, te, act: (t, 0)),
                pl.BlockSpec((_TBS, 1), lambda f, t, te, act: (t, 0)),
                pl.BlockSpec((1, d, _FB), lambda f, t, te, act: (te[t], 0, f)),
                pl.BlockSpec((1, d, _FB), lambda f, t, te, act: (te[t], 0, f)),
                pl.BlockSpec((1, _FB, d), lambda f, t, te, act: (te[t], f, 0)),
            ],
            out_specs=pl.BlockSpec((1, _TBS, d),
                                   lambda f, t, te, act: (f, t, 0)),
            scratch_shapes=[
                pltpu.VMEM((d, _FB), jnp.bfloat16),
                pltpu.VMEM((d, _FB), jnp.bfloat16),
                pltpu.VMEM((_FB, d), jnp.bfloat16),
            ],
        ),
        out_shape=jax.ShapeDtypeStruct((nf, p_rows, d), jnp.float32),
        compiler_params=pltpu.CompilerParams(
            dimension_semantics=("arbitrary", "arbitrary")),
    )(tile_e, tile_act, xs, w_sorted, gate_proj, up_proj, down_proj)

    eof = eo.reshape(nf * p_rows, d)
    plane = jnp.arange(nf, dtype=jnp.int32)[:, None] * p_rows
    idx4 = (jnp.stack([pos[0::2], pos[1::2]])[None, :, :]
            + plane[:, :, None]).reshape(2 * nf, t_tok)
    ct = 8
    idx4r = (idx4.reshape(2 * nf, _NW, t_tok // (_NW * ct), ct)
             .transpose(1, 2, 0, 3)
             .reshape(t_tok // ct, 2 * nf, ct))

    out = _sc_combine(xt, eof, idx4r, t_tok, d)
    return out.reshape(b, s, d)


# speedup vs baseline: 1.2252x; 1.0499x over previous
import functools

import jax
import jax.numpy as jnp
from jax import lax
from jax.experimental import pallas as pl
from jax.experimental.pallas import tpu as pltpu
from jax.experimental.pallas import tpu_sc as plsc

_EPS = 1e-6
_TB = 256
_TBS = 256
_FB = 512
_NW = 32


def _router_body(xt_ref, rms_ref, rw_ref, rb_ref, xn_ref, idx_ref, aff_ref):
    x = xt_ref[:]
    var = jnp.mean(x * x, axis=1, keepdims=True)
    xn = (x * lax.rsqrt(var + _EPS)) * rms_ref[:]
    xn_ref[:] = xn
    logits = jnp.dot(xn, rw_ref[:], preferred_element_type=jnp.float32)
    scores = jax.nn.sigmoid(logits)
    choice = scores + rb_ref[:]
    iota = lax.broadcasted_iota(jnp.int32, (_TB, 128), 1)
    m1 = jnp.max(choice, axis=1, keepdims=True)
    i1 = jnp.min(jnp.where(choice == m1, iota, 128), axis=1, keepdims=True)
    mask1 = iota == i1
    choice2 = jnp.where(mask1, -jnp.inf, choice)
    m2 = jnp.max(choice2, axis=1, keepdims=True)
    i2 = jnp.min(jnp.where(choice2 == m2, iota, 128), axis=1, keepdims=True)
    mask2 = iota == i2
    a1 = jnp.sum(jnp.where(mask1, scores, 0.0), axis=1, keepdims=True)
    a2 = jnp.sum(jnp.where(mask2, scores, 0.0), axis=1, keepdims=True)
    den = a1 + a2 + 1e-9
    idx_ref[:] = jnp.concatenate([i1, i2], axis=1)
    aff_ref[:] = jnp.concatenate([a1 / den, a2 / den], axis=1)


def _gemm_body(te_ref, act_ref, xs_ref, wc_ref, g_ref, u_ref, d_ref,
               eo_ref, gbf, ubf, dbf):
    t = pl.program_id(1)

    @pl.when(act_ref[t] == 1)
    def _compute():
        prev = te_ref[jnp.maximum(t - 1, 0)]

        @pl.when(jnp.logical_or(t == 0, prev != te_ref[t]))
        def _cast_weights():
            gbf[:] = g_ref[0].astype(jnp.bfloat16)
            ubf[:] = u_ref[0].astype(jnp.bfloat16)
            dbf[:] = d_ref[0].astype(jnp.bfloat16)

        xsb = xs_ref[:].astype(jnp.bfloat16)
        g = jnp.dot(xsb, gbf[:], preferred_element_type=jnp.float32)
        u = jnp.dot(xsb, ubf[:], preferred_element_type=jnp.float32)
        h = (g * jax.nn.sigmoid(g)) * u
        o = jnp.dot(h.astype(jnp.bfloat16), dbf[:],
                    preferred_element_type=jnp.float32)
        eo_ref[0] = o * wc_ref[:]


def _sc_gather(xn, row_token, p_rows, d):
    rows_w = p_rows // _NW
    chunk = 8
    nch = rows_w // chunk
    mesh = plsc.VectorSubcoreMesh(core_axis_name="c", subcore_axis_name="s")

    nslots = 6
    @functools.partial(
        pl.kernel, mesh=mesh,
        out_type=jax.ShapeDtypeStruct((p_rows, d), jnp.float32),
        scratch_types=[
            pltpu.VMEM((rows_w,), jnp.int32),
            pltpu.VMEM((nslots, chunk, d), jnp.float32),
        ] + [pltpu.SemaphoreType.DMA] * nslots,
    )
    def gk(xn_hbm, rt_hbm, xs_hbm, idx_v, rows_v, *sems):
        wid = lax.axis_index("s") * 2 + lax.axis_index("c")
        base = wid * rows_w
        pltpu.sync_copy(rt_hbm.at[pl.ds(base, rows_w)], idx_v)

        def fire_gather(ch):
            return pltpu.async_copy(
                xn_hbm.at[idx_v.at[pl.ds(ch * chunk, chunk)]],
                rows_v.at[ch % nslots], sems[ch % nslots])

        def fire_store(ch):
            return pltpu.async_copy(
                rows_v.at[ch % nslots],
                xs_hbm.at[pl.ds(base + ch * chunk, chunk)],
                sems[ch % nslots])

        gs = {ch: fire_gather(ch) for ch in range(min(nslots, nch))}
        ss = {}
        for ch in range(nch):
            gs[ch].wait()
            ss[ch] = fire_store(ch)
            nxt = ch + nslots
            if nxt < nch:
                ss[ch].wait()
                gs[nxt] = fire_gather(nxt)
        for ch in range(max(0, nch - nslots), nch):
            ss[ch].wait()

    return gk(xn, row_token)


def _sc_combine(xt, eof, idx4r, t_tok, d):
    tok_w = t_tok // _NW
    ct = 4
    nch = tok_w // ct
    nj = idx4r.shape[1]
    mesh = plsc.VectorSubcoreMesh(core_axis_name="c", subcore_axis_name="s")

    @functools.partial(
        pl.kernel, mesh=mesh,
        out_type=jax.ShapeDtypeStruct((t_tok, d), jnp.float32),
        scratch_types=[
            pltpu.VMEM((2, ct, d), jnp.float32),
            pltpu.VMEM((2, nj, ct, d), jnp.float32),
            pltpu.VMEM((nch, nj, ct), jnp.int32),
            pltpu.SemaphoreType.DMA,
            pltpu.SemaphoreType.DMA,
            pltpu.SemaphoreType.DMA,
            pltpu.SemaphoreType.DMA,
        ],
    )
    def ck(xt_hbm, eof_hbm, idx_hbm, out_hbm, acc_v, bufs_v, idx_v,
           sem0, sem1, osem0, osem1):
        wid = lax.axis_index("s") * 2 + lax.axis_index("c")
        sems = (sem0, sem1)
        osems = (osem0, osem1)
        pltpu.sync_copy(idx_hbm.at[pl.ds(wid * nch, nch)], idx_v)

        def fire(ch):
            s = ch % 2
            base = wid * tok_w + ch * ct
            cps = [pltpu.async_copy(
                eof_hbm.at[idx_v.at[ch, j]], bufs_v.at[s, j], sems[s])
                for j in range(nj)]
            cps.append(pltpu.async_copy(
                xt_hbm.at[pl.ds(base, ct)], acc_v.at[s], sems[s]))
            return cps

        ost = {}
        inflight = {0: fire(0)}
        for ch in range(nch):
            s = ch % 2
            if ch + 1 < nch:
                if ch - 1 in ost:
                    ost[ch - 1].wait()
                inflight[ch + 1] = fire(ch + 1)
            for cp in inflight.pop(ch):
                cp.wait()

            def addbody(i, _):
                r = i // (d // 64)
                c = (i % (d // 64)) * 64
                for q in range(4):
                    cq = c + q * 16
                    acc_v[s, r, pl.ds(cq, 16)] = (
                        ((acc_v[s, r, pl.ds(cq, 16)]
                          + bufs_v[s, 0, r, pl.ds(cq, 16)])
                         + (bufs_v[s, 1, r, pl.ds(cq, 16)]
                            + bufs_v[s, 2, r, pl.ds(cq, 16)]))
                        + bufs_v[s, 3, r, pl.ds(cq, 16)])
                return 0

            lax.fori_loop(0, ct * (d // 64), addbody, 0)
            ost[ch] = pltpu.async_copy(
                acc_v.at[s], out_hbm.at[pl.ds(wid * tok_w + ch * ct, ct)],
                osems[s])
        for ch in (nch - 2, nch - 1):
            if ch in ost:
                ost[ch].wait()

    return ck(xt, eof, idx4r)


def kernel(hidden_states, rms_weight, router_weight, router_bias,
           gate_proj, up_proj, down_proj):
    b, s, d = hidden_states.shape
    e_num, _, f_dim = gate_proj.shape
    t_tok = b * s
    nt = t_tok // _TB
    nf = f_dim // _FB
    n_pairs = 2 * t_tok
    ntiles = (n_pairs + e_num * _TBS) // _TBS
    p_rows = ntiles * _TBS
    xt = hidden_states.reshape(t_tok, d)
    rw_pad = jnp.zeros((d, 128), jnp.float32).at[:, :e_num].set(router_weight.T)
    rb_pad = jnp.full((1, 128), -1e30, jnp.float32).at[0, :e_num].set(router_bias)
    rms2 = rms_weight.reshape(1, d)

    xn, idx, aff = pl.pallas_call(
        _router_body,
        grid=(nt,),
        in_specs=[
            pl.BlockSpec((_TB, d), lambda t: (t, 0)),
            pl.BlockSpec((1, d), lambda t: (0, 0)),
            pl.BlockSpec((d, 128), lambda t: (0, 0)),
            pl.BlockSpec((1, 128), lambda t: (0, 0)),
        ],
        out_specs=[
            pl.BlockSpec((_TB, d), lambda t: (t, 0)),
            pl.BlockSpec((_TB, 2), lambda t: (t, 0)),
            pl.BlockSpec((_TB, 2), lambda t: (t, 0)),
        ],
        out_shape=[
            jax.ShapeDtypeStruct((t_tok, d), jnp.float32),
            jax.ShapeDtypeStruct((t_tok, 2), jnp.int32),
            jax.ShapeDtypeStruct((t_tok, 2), jnp.float32),
        ],
    )(xt, rms2, rw_pad, rb_pad)

    flat_e = idx.reshape(n_pairs)
    flat_aff = aff.reshape(n_pairs)
    onehot = (flat_e[:, None] == jnp.arange(e_num)[None, :]).astype(jnp.int32)
    counts = jnp.sum(onehot, axis=0)
    rank = jnp.take_along_axis(jnp.cumsum(onehot, axis=0) - onehot,
                               flat_e[:, None], axis=1)[:, 0]
    p_pad = ((counts + _TBS - 1) // _TBS) * _TBS
    starts = jnp.concatenate([jnp.zeros(1, p_pad.dtype),
                              jnp.cumsum(p_pad)[:-1]])
    pos = (starts[flat_e] + rank).astype(jnp.int32)
    row_token = jnp.zeros(p_rows, jnp.int32).at[pos].set(
        jnp.arange(n_pairs, dtype=jnp.int32) // 2)
    w_sorted = jnp.zeros((p_rows, 1), jnp.float32).at[pos, 0].set(flat_aff)
    n_used = (jnp.sum(p_pad) // _TBS).astype(jnp.int32)
    tile_ids = jnp.arange(ntiles, dtype=jnp.int32)
    tile_e = jnp.clip(
        jnp.searchsorted(starts, tile_ids * _TBS, side='right') - 1,
        0, e_num - 1).astype(jnp.int32)
    tile_act = (tile_ids < n_used).astype(jnp.int32)
    te_last = tile_e[jnp.clip(n_used - 1, 0, ntiles - 1)]
    tile_e = jnp.where(tile_act == 1, tile_e, te_last)

    xs = _sc_gather(xn, row_token, p_rows, d)

    eo = pl.pallas_call(
        _gemm_body,
        grid_spec=pltpu.PrefetchScalarGridSpec(
            num_scalar_prefetch=2,
            grid=(nf, ntiles),
            in_specs=[
                pl.BlockSpec((_TBS, d), lambda f, t, te, act: (t, 0)),
                pl.BlockSpec((_TBS, 1), lambda f, t, te, act: (t, 0)),
                pl.BlockSpec((1, d, _FB), lambda f, t, te, act: (te[t], 0, f)),
                pl.BlockSpec((1, d, _FB), lambda f, t, te, act: (te[t], 0, f)),
                pl.BlockSpec((1, _FB, d), lambda f, t, te, act: (te[t], f, 0)),
            ],
            out_specs=pl.BlockSpec((1, _TBS, d),
                                   lambda f, t, te, act: (f, t, 0)),
            scratch_shapes=[
                pltpu.VMEM((d, _FB), jnp.bfloat16),
                pltpu.VMEM((d, _FB), jnp.bfloat16),
                pltpu.VMEM((_FB, d), jnp.bfloat16),
            ],
        ),
        out_shape=jax.ShapeDtypeStruct((nf, p_rows, d), jnp.float32),
        compiler_params=pltpu.CompilerParams(
            dimension_semantics=("arbitrary", "arbitrary")),
    )(tile_e, tile_act, xs, w_sorted, gate_proj, up_proj, down_proj)

    eof = eo.reshape(nf * p_rows, d)
    plane = jnp.arange(nf, dtype=jnp.int32)[:, None] * p_rows
    idx4 = (jnp.stack([pos[0::2], pos[1::2]])[None, :, :]
            + plane[:, :, None]).reshape(2 * nf, t_tok)
    ct = 4
    idx4r = (idx4.reshape(2 * nf, _NW, t_tok // (_NW * ct), ct)
             .transpose(1, 2, 0, 3)
             .reshape(t_tok // ct, 2 * nf, ct))

    out = _sc_combine(xt, eof, idx4r, t_tok, d)
    return out.reshape(b, s, d)
